# baseline (speedup 1.0000x reference)
import jax
import jax.numpy as jnp
from jax import lax
from jax.experimental import pallas as pl
from jax.experimental.pallas import tpu as pltpu
from jax.experimental.pallas import tpu_sc as plsc

NUM_SLOTS = 64
OUT_SIDE = 1024
OUT_ELEMS = OUT_SIDE * OUT_SIDE
NC = 2
NS = 16
LANES = 16
NW = NC * NS
CHUNK = OUT_ELEMS // NW


def _sc_body(wb_hbm, param_hbm, out_hbm, wb_v, buf0, buf1, acc, sem0, sem1):
    wid = lax.axis_index("s") * NC + lax.axis_index("c")
    base = wid * CHUNK
    pltpu.sync_copy(wb_hbm, wb_v)
    bufs = (buf0, buf1)
    sems = (sem0, sem1)
    copies = [None, None]
    copies[0] = pltpu.async_copy(
        param_hbm.at[0, pl.ds(base, CHUNK)], bufs[0], sems[0])
    for s in range(NUM_SLOTS):
        if s + 1 < NUM_SLOTS:
            copies[(s + 1) % 2] = pltpu.async_copy(
                param_hbm.at[s + 1, pl.ds(base, CHUNK)],
                bufs[(s + 1) % 2], sems[(s + 1) % 2])
        copies[s % 2].wait()
        buf = bufs[s % 2]
        wv = wb_v[s]
        first = s == 0
        unroll = 8
        def body(i, _, buf=buf, wv=wv, first=first):
            for u in range(unroll):
                sl = pl.ds((i * unroll + u) * LANES, LANES)
                x = wv * buf[sl]
                acc[sl] = x if first else acc[sl] + x
            return 0
        lax.fori_loop(0, CHUNK // (LANES * unroll), body, 0)
    pltpu.sync_copy(acc, out_hbm.at[pl.ds(base, CHUNK)])


def kernel(superposition_weights, parameter):
    wb = jnp.broadcast_to(
        superposition_weights[:, None], (NUM_SLOTS, LANES))
    pflat = parameter.reshape(NUM_SLOTS, OUT_ELEMS)
    run = pl.kernel(
        _sc_body,
        out_type=jax.ShapeDtypeStruct((OUT_ELEMS,), jnp.float32),
        mesh=plsc.VectorSubcoreMesh(
            core_axis_name="c", subcore_axis_name="s"),
        scratch_types=[
            pltpu.VMEM((NUM_SLOTS, LANES), jnp.float32),
            pltpu.VMEM((CHUNK,), jnp.float32),
            pltpu.VMEM((CHUNK,), jnp.float32),
            pltpu.VMEM((CHUNK,), jnp.float32),
            pltpu.SemaphoreType.DMA,
            pltpu.SemaphoreType.DMA,
        ],
    )
    out = run(wb, pflat)
    return out.reshape(OUT_SIDE, OUT_SIDE)


# speedup vs baseline: 1.0026x; 1.0026x over previous
import jax
import jax.numpy as jnp
from jax import lax
from jax.experimental import pallas as pl
from jax.experimental.pallas import tpu as pltpu
from jax.experimental.pallas import tpu_sc as plsc

NUM_SLOTS = 64
OUT_SIDE = 1024
OUT_ELEMS = OUT_SIDE * OUT_SIDE
NC = 2
NS = 16
LANES = 16
NW = NC * NS
CHUNK = OUT_ELEMS // NW


def _sc_body(wb_hbm, param_hbm, out_hbm, wb_v, buf0, buf1, acc, sem0, sem1):
    wid = lax.axis_index("s") * NC + lax.axis_index("c")
    base = wid * CHUNK
    pltpu.sync_copy(wb_hbm, wb_v)
    bufs = (buf0, buf1)
    sems = (sem0, sem1)
    copies = [None, None]
    copies[0] = pltpu.async_copy(
        param_hbm.at[0, pl.ds(base, CHUNK)], bufs[0], sems[0])
    for s in range(NUM_SLOTS):
        if s + 1 < NUM_SLOTS:
            copies[(s + 1) % 2] = pltpu.async_copy(
                param_hbm.at[s + 1, pl.ds(base, CHUNK)],
                bufs[(s + 1) % 2], sems[(s + 1) % 2])
        copies[s % 2].wait()
        buf = bufs[s % 2]
        wv = wb_v[s]
        if s == 0:
            def body0(i, buf=buf, wv=wv):
                sl = pl.ds(i * LANES, LANES)
                acc[sl] = wv * buf[sl]
            plsc.parallel_loop(0, CHUNK // LANES, 1, unroll=8)(body0)
        else:
            def body(i, buf=buf, wv=wv):
                sl = pl.ds(i * LANES, LANES)
                plsc.addupdate(acc.at[sl], wv * buf[sl])
            plsc.parallel_loop(0, CHUNK // LANES, 1, unroll=8)(body)
    pltpu.sync_copy(acc, out_hbm.at[pl.ds(base, CHUNK)])


def kernel(superposition_weights, parameter):
    wb = jnp.broadcast_to(
        superposition_weights[:, None], (NUM_SLOTS, LANES))
    pflat = parameter.reshape(NUM_SLOTS, OUT_ELEMS)
    run = pl.kernel(
        _sc_body,
        out_type=jax.ShapeDtypeStruct((OUT_ELEMS,), jnp.float32),
        mesh=plsc.VectorSubcoreMesh(
            core_axis_name="c", subcore_axis_name="s"),
        scratch_types=[
            pltpu.VMEM((NUM_SLOTS, LANES), jnp.float32),
            pltpu.VMEM((CHUNK,), jnp.float32),
            pltpu.VMEM((CHUNK,), jnp.float32),
            pltpu.VMEM((CHUNK,), jnp.float32),
            pltpu.SemaphoreType.DMA,
            pltpu.SemaphoreType.DMA,
        ],
    )
    out = run(wb, pflat)
    return out.reshape(OUT_SIDE, OUT_SIDE)


# speedup vs baseline: 2.0802x; 2.0749x over previous
import jax
import jax.numpy as jnp
from jax import lax
from jax.experimental import pallas as pl
from jax.experimental.pallas import tpu as pltpu
from jax.experimental.pallas import tpu_sc as plsc

NUM_SLOTS = 64
OUT_SIDE = 1024
NC = 2
NS = 16
LANES = 16
NW = NC * NS
ROWS = OUT_SIDE // NW
CHUNK = ROWS * OUT_SIDE
NVEC = CHUNK // LANES
CPR = OUT_SIDE // LANES


def _sc_body(wb_hbm, param_hbm, out_hbm, wb_v, buf0, buf1, acc, sem0, sem1):
    wid = lax.axis_index("s") * NC + lax.axis_index("c")
    row0 = wid * ROWS
    pltpu.sync_copy(wb_hbm, wb_v)
    bufs = (buf0, buf1)
    sems = (sem0, sem1)
    copies = [None, None]
    copies[0] = pltpu.async_copy(
        param_hbm.at[0, pl.ds(row0, ROWS), :], bufs[0], sems[0])
    for s in range(NUM_SLOTS):
        if s + 1 < NUM_SLOTS:
            copies[(s + 1) % 2] = pltpu.async_copy(
                param_hbm.at[s + 1, pl.ds(row0, ROWS), :],
                bufs[(s + 1) % 2], sems[(s + 1) % 2])
        copies[s % 2].wait()
        buf = bufs[s % 2]
        wv = wb_v[s, pl.ds(0, LANES)]
        if s == 0:
            def body0(i, buf=buf, wv=wv):
                r = i // CPR
                c = (i % CPR) * LANES
                acc[r, pl.ds(c, LANES)] = wv * buf[r, pl.ds(c, LANES)]
            plsc.parallel_loop(0, NVEC, 1, unroll=8)(body0)
        else:
            def body(i, buf=buf, wv=wv):
                r = i // CPR
                c = (i % CPR) * LANES
                plsc.addupdate(
                    acc.at[r, pl.ds(c, LANES)], wv * buf[r, pl.ds(c, LANES)])
            plsc.parallel_loop(0, NVEC, 1, unroll=8)(body)
    pltpu.sync_copy(acc, out_hbm.at[pl.ds(row0, ROWS), :])


def kernel(superposition_weights, parameter):
    wb = jnp.broadcast_to(
        superposition_weights[:, None], (NUM_SLOTS, 128))
    run = pl.kernel(
        _sc_body,
        out_type=jax.ShapeDtypeStruct((OUT_SIDE, OUT_SIDE), jnp.float32),
        mesh=plsc.VectorSubcoreMesh(
            core_axis_name="c", subcore_axis_name="s"),
        compiler_params=pltpu.CompilerParams(use_tc_tiling_on_sc=True),
        scratch_types=[
            pltpu.VMEM((NUM_SLOTS, 128), jnp.float32),
            pltpu.VMEM((ROWS, OUT_SIDE), jnp.float32),
            pltpu.VMEM((ROWS, OUT_SIDE), jnp.float32),
            pltpu.VMEM((ROWS, OUT_SIDE), jnp.float32),
            pltpu.SemaphoreType.DMA,
            pltpu.SemaphoreType.DMA,
        ],
    )
    return run(wb, parameter)
